# Initial kernel scaffold; baseline (speedup 1.0000x reference)
import jax
import jax.numpy as jnp
from jax.experimental import pallas as pl


def kernel(scores, H, A, pooling_size):
    raise NotImplementedError("write your pallas kernel here")


# speedup vs baseline: 2.3076x; 2.3076x over previous
import jax
import jax.numpy as jnp
from jax import lax
from jax.experimental import pallas as pl
from jax.experimental.pallas import tpu as pltpu
from jax.experimental.pallas import tpu_sc as plsc

N = 10000
K = 5000
D = 128
KPAD = 5120
NFULL = 312
TAIL = K - 16
G_SMALL = 19
W_BIG = 17


def _sc_gather_body(a_hbm, h_hbm, idx_hbm, out_a, out_h,
                    colidx_v, rbuf, obuf, hbuf):
    cid = lax.axis_index("c")
    sid = lax.axis_index("s")
    w = sid * 2 + cid
    g0 = w * G_SMALL + jnp.minimum(w, W_BIG)
    gcount = G_SMALL + (w < W_BIG).astype(jnp.int32)

    pltpu.sync_copy(idx_hbm, colidx_v)

    def group_body(gl, carry):
        rg = (g0 + gl) * 8
        idx8 = colidx_v.at[pl.ds(rg, 8)]
        pltpu.sync_copy(a_hbm.at[idx8], rbuf)
        pltpu.sync_copy(h_hbm.at[idx8], hbuf)
        for r in range(8):
            rsp = lax.full((16,), r, jnp.int32)

            def chunk_body(j, c, r=r, rsp=rsp):
                cidx = colidx_v[pl.ds(j * 16, 16)]
                obuf[r, pl.ds(j * 16, 16)] = plsc.load_gather(rbuf, [rsp, cidx])
                return c

            lax.fori_loop(0, NFULL, chunk_body, 0, unroll=8)
            cidx = colidx_v[pl.ds(TAIL, 16)]
            obuf[r, pl.ds(TAIL, 16)] = plsc.load_gather(rbuf, [rsp, cidx])
        pltpu.sync_copy(obuf, out_a.at[pl.ds(rg, 8)])
        pltpu.sync_copy(hbuf, out_h.at[pl.ds(rg, 8)])
        return carry

    lax.fori_loop(0, gcount, group_body, 0)


def _scale_body(h_ref, v_ref, o_ref):
    o_ref[...] = h_ref[...] * v_ref[...]


def _scale_rows(h_raw, values):
    return pl.pallas_call(
        _scale_body,
        grid=(5,),
        in_specs=[
            pl.BlockSpec((K // 5, D), lambda i: (i, 0)),
            pl.BlockSpec((K // 5, 1), lambda i: (i, 0)),
        ],
        out_specs=pl.BlockSpec((K // 5, D), lambda i: (i, 0)),
        out_shape=jax.ShapeDtypeStruct((K, D), jnp.float32),
    )(h_raw, values.reshape(K, 1))


def kernel(scores, H, A, pooling_size):
    del pooling_size
    values, idx = lax.top_k(scores, K)
    idx = idx.astype(jnp.int32)
    idx_pad = jnp.concatenate([idx, jnp.zeros((KPAD - K,), jnp.int32)])

    mesh = plsc.VectorSubcoreMesh(core_axis_name="c", subcore_axis_name="s")
    pooled_A, h_raw = pl.kernel(
        _sc_gather_body,
        out_type=[
            jax.ShapeDtypeStruct((K, K), jnp.float32),
            jax.ShapeDtypeStruct((K, D), jnp.float32),
        ],
        mesh=mesh,
        compiler_params=pltpu.CompilerParams(
            needs_layout_passes=False, use_tc_tiling_on_sc=False),
        scratch_types=[
            pltpu.VMEM((KPAD,), jnp.int32),
            pltpu.VMEM((8, N), jnp.float32),
            pltpu.VMEM((8, K), jnp.float32),
            pltpu.VMEM((8, D), jnp.float32),
        ],
    )(A, H, idx_pad)
    pooled_H = _scale_rows(h_raw, values)
    return (pooled_H, pooled_A, idx)
